# baseline (speedup 1.0000x reference)
import functools

import jax
import jax.numpy as jnp
from jax import lax
from jax.experimental import pallas as pl
from jax.experimental.pallas import tpu as pltpu
from jax.experimental.pallas import tpu_sc as plsc

G = 64
NSC = 2
NSUB = 16
EK = 80


def _make_sc_agg(N, E, half, edge_split=False):
    nworkers = NSC * NSUB if edge_split else NSUB
    epw = E // nworkers
    rows = (N // NSUB) // 8 * 8
    tail = N - NSUB * rows
    n_chunks = epw // EK
    mesh = plsc.VectorSubcoreMesh(core_axis_name="c", subcore_axis_name="s")

    NG = 4
    NI = 8
    scratch = (
        [pltpu.VMEM((EK,), jnp.int32) for _ in range(NI)]
        + [pltpu.VMEM((EK,), jnp.int32) for _ in range(NI)]
        + [pltpu.VMEM((EK, half), jnp.float32) for _ in range(NG)]
        + [pltpu.VMEM_SHARED((N, half), jnp.float32)]
        + [pltpu.SemaphoreType.DMA for _ in range(2 * NI + NG + 2)]
    )

    @functools.partial(
        pl.kernel,
        out_type=jax.ShapeDtypeStruct((2 * N, half), jnp.float32),
        mesh=mesh,
        scratch_types=scratch,
    )
    def sc_agg(table, srcs2, dst, zeros, out, *sc_args):
        src_v = sc_args[:NI]
        dst_v = sc_args[NI:2 * NI]
        rows_bufs = sc_args[2 * NI:2 * NI + NG]
        acc = sc_args[2 * NI + NG]
        sems = sc_args[2 * NI + NG + 1:]
        sems_s = sems[:NI]
        sems_d = sems[NI:2 * NI]
        sems_g = sems[2 * NI:2 * NI + NG]
        sems_sc = sems[2 * NI + NG:]

        c = lax.axis_index("c")
        s = lax.axis_index("s")
        if edge_split:
            sbase = (c * NSUB + s) * epw
            dbase = sbase
        else:
            sbase = c * E + s * epw
            dbase = s * epw

        def src_ref(i):
            return srcs2.at[pl.ds(sbase + i * EK, EK)]

        def dst_ref(i):
            return dst.at[pl.ds(dbase + i * EK, EK)]

        def issue_idx(j, slot):
            pltpu.async_copy(src_ref(j), src_v[slot], sems_s[slot])
            pltpu.async_copy(dst_ref(j), dst_v[slot], sems_d[slot])

        def issue_gather(j, slot, gb):
            pltpu.make_async_copy(src_ref(j), src_v[slot],
                                  sems_s[slot]).wait()
            pltpu.async_copy(table.at[src_v[slot]], rows_bufs[gb],
                             sems_g[gb])

        for j in range(6):
            issue_idx(j, j)
        pltpu.sync_copy(zeros.at[pl.ds(s * rows, rows)],
                        acc.at[pl.ds(s * rows, rows)])

        @pl.when(s == NSUB - 1)
        def _zero_tail():
            pltpu.sync_copy(zeros.at[pl.ds(NSUB * rows, tail)],
                            acc.at[pl.ds(NSUB * rows, tail)])

        issue_gather(0, 0, 0)
        issue_gather(1, 1, 1)
        issue_gather(2, 2, 2)
        plsc.subcore_barrier()

        def step(i, b, static_i=None):
            gb = b % NG
            pltpu.make_async_copy(table.at[src_v[b % NI]], rows_bufs[gb],
                                  sems_g[gb]).wait()

            def guarded(cond, fn):
                if static_i is None:
                    pl.when(cond)(fn)
                elif cond:
                    fn()

            def wait_prev_scatter():
                pltpu.make_async_copy(
                    rows_bufs[(b - 1) % NG], acc.at[dst_v[(b - 1) % NI]],
                    sems_sc[(b - 1) % 2]).wait()
            if static_i is None and b == 0:
                guarded(i >= 1, wait_prev_scatter)
            elif static_i is None or static_i >= 1:
                wait_prev_scatter()

            guarded(i + 6 < n_chunks,
                    lambda: issue_idx(i + 6, (b + 6) % NI))
            guarded(i + 3 < n_chunks,
                    lambda: issue_gather(i + 3, (b + 3) % NI, (b + 3) % NG))

            pltpu.make_async_copy(dst_ref(i), dst_v[b % NI],
                                  sems_d[b % NI]).wait()
            pltpu.async_copy(rows_bufs[gb], acc.at[dst_v[b % NI]],
                             sems_sc[b % 2], add=True)

        GRP = NI

        def group(g, carry):
            for b in range(GRP):
                step(g * GRP + b, b)
            return carry

        n_groups = n_chunks // GRP
        lax.fori_loop(0, n_groups, group, 0)
        for i in range(n_groups * GRP, n_chunks):
            step(i, i % GRP, static_i=i)
        pltpu.make_async_copy(
            rows_bufs[(n_chunks - 1) % NG],
            acc.at[dst_v[(n_chunks - 1) % NI]],
            sems_sc[(n_chunks - 1) % 2]).wait()
        plsc.subcore_barrier()
        pltpu.sync_copy(acc.at[pl.ds(s * rows, rows)],
                        out.at[pl.ds(c * N + s * rows, rows)])

        @pl.when(s == NSUB - 1)
        def _write_tail():
            pltpu.sync_copy(acc.at[pl.ds(NSUB * rows, tail)],
                            out.at[pl.ds(c * N + NSUB * rows, tail)])

    return sc_agg


def _dot3(a, b):
    ah = a.astype(jnp.bfloat16)
    al = (a - ah.astype(jnp.float32)).astype(jnp.bfloat16)
    bh = b.astype(jnp.bfloat16)
    bl = (b - bh.astype(jnp.float32)).astype(jnp.bfloat16)
    f = jnp.float32
    return (jnp.dot(ah, bh, preferred_element_type=f)
            + (jnp.dot(al, bh, preferred_element_type=f)
               + jnp.dot(ah, bl, preferred_element_type=f)))


def _dotsel(oh, m):
    ohb = oh.astype(jnp.bfloat16)
    mh = m.astype(jnp.bfloat16)
    ml = (m - mh.astype(jnp.float32)).astype(jnp.bfloat16)
    f = jnp.float32
    return (jnp.dot(ohb, mh, preferred_element_type=f)
            + jnp.dot(ohb, ml, preferred_element_type=f))


def _layer_body(h_ref, agg_ref, batch_ref, w1, b1, w2, b2, gw, gb, gm,
                out_ref, hmlp, ssum, ssq, scnt, *, R, H, final,
                agg_partial=False, head=None, psum=None, NB=None):
    p = pl.program_id(0)
    i = pl.program_id(1)

    @pl.when(jnp.logical_and(p == 0, i == 0))
    def _init():
        ssum[...] = jnp.zeros_like(ssum)
        ssq[...] = jnp.zeros_like(ssq)
        scnt[...] = jnp.zeros_like(scnt)
        if final:
            psum[...] = jnp.zeros_like(psum)

    b = batch_ref[0, 0, :]
    onehot = (b[:, None] == lax.broadcasted_iota(jnp.int32, (R, G), 1)
              ).astype(jnp.float32)

    @pl.when(p == 0)
    def _phase0():
        if agg_partial:
            hin = h_ref[...] + agg_ref[0] + agg_ref[1]
        else:
            hin = jnp.concatenate(
                [h_ref[0] + agg_ref[0], h_ref[1] + agg_ref[1]], axis=1)
        t = jnp.maximum(_dot3(hin, w1[...]) + b1[...], 0.0)
        t = _dot3(t, w2[...]) + b2[...]
        hmlp[pl.ds(i * R, R), :] = t
        dn = (((0,), (0,)), ((), ()))
        ssum[...] += lax.dot_general(onehot, t, dn,
                                     preferred_element_type=jnp.float32,
                precision=lax.Precision.DEFAULT)
        ssq[...] += lax.dot_general(onehot, t * t, dn,
                                    preferred_element_type=jnp.float32,
                precision=lax.Precision.DEFAULT)
        scnt[...] += jnp.broadcast_to(jnp.sum(onehot, axis=0)[:, None], (G, H))

    @pl.when(p == 1)
    def _phase1():
        cnt = jnp.maximum(scnt[...], 1.0)
        mean = ssum[...] / cnt
        m2 = ssq[...] / cnt
        gmv = gm[...]
        var = m2 - mean * mean * gmv * (2.0 - gmv)
        rstd = lax.rsqrt(var + 1e-5)
        mn = _dotsel(onehot, mean * gmv)
        rs = _dotsel(onehot, gw[...] * rstd)
        y = (hmlp[pl.ds(i * R, R), :] - mn) * rs + gb[...]
        y = jnp.maximum(y, 0.0)
        if not final:
            out_ref[...] = jnp.stack([y[:, :128], y[:, 128:]], axis=0)
        else:
            dn = (((0,), (0,)), ((), ()))
            psum[...] += lax.dot_general(onehot, y, dn,
                                         preferred_element_type=jnp.float32,
                precision=lax.Precision.DEFAULT)

            @pl.when(i == NB - 1)
            def _head():
                wm1, bm1, wm2, bm2 = head
                pooled = psum[...] / cnt
                z = jnp.maximum(
                    jnp.dot(pooled, wm1[...],
                            preferred_element_type=jnp.float32,
                precision=lax.Precision.HIGHEST) + bm1[...],
                    0.0)
                z = jnp.dot(z, wm2[...],
                            preferred_element_type=jnp.float32,
                precision=lax.Precision.HIGHEST) + bm2[...]
                out_ref[...] = z


def _make_tc_layer(N, fin, H, R, final, C=None, agg_partial=False):
    NB = N // R
    half = fin // 2
    agg_w = fin if agg_partial else half
    grid = (2, NB)

    def wspec(shape):
        return pl.BlockSpec(shape, lambda p, i: (0,) * len(shape))

    p0blk = lambda p, i: (0, jnp.where(p == 0, i, 0), 0)
    if agg_partial:
        h_spec = pl.BlockSpec((R, fin), lambda p, i: (jnp.where(p == 0, i, 0),
                                                      0))
    else:
        h_spec = pl.BlockSpec((2, R, half), p0blk)
    in_specs = [
        h_spec,
        pl.BlockSpec((2, R, agg_w), p0blk),
        pl.BlockSpec((1, 1, R), lambda p, i: (i, 0, 0)),
        wspec((fin, H)), wspec((1, H)), wspec((H, H)), wspec((1, H)),
        wspec((1, H)), wspec((1, H)), wspec((1, H)),
    ]
    scratch = [
        pltpu.VMEM((N, H), jnp.float32),
        pltpu.VMEM((G, H), jnp.float32),
        pltpu.VMEM((G, H), jnp.float32),
        pltpu.VMEM((G, H), jnp.float32),
    ]
    if final:
        in_specs += [wspec((H, H)), wspec((1, H)), wspec((H, C)),
                     wspec((1, C))]
        out_spec = pl.BlockSpec((G, C), lambda p, i: (0, 0))
        out_shape = jax.ShapeDtypeStruct((G, C), jnp.float32)
        scratch.append(pltpu.VMEM((G, H), jnp.float32))

        def body(h, a, bt, w1, b1, w2, b2, gw, gb, gm, wm1, bm1, wm2, bm2,
                 out_ref, hmlp, ssum, ssq, scnt, psum):
            _layer_body(h, a, bt, w1, b1, w2, b2, gw, gb, gm, out_ref,
                        hmlp, ssum, ssq, scnt, R=R, H=H, final=True,
                        agg_partial=agg_partial,
                        head=(wm1, bm1, wm2, bm2), psum=psum, NB=NB)
    else:
        out_spec = pl.BlockSpec((2, R, 128),
                                lambda p, i: (0, jnp.where(p == 0, 0, i), 0))
        out_shape = jax.ShapeDtypeStruct((2, N, 128), jnp.float32)

        def body(h, a, bt, w1, b1, w2, b2, gw, gb, gm,
                 out_ref, hmlp, ssum, ssq, scnt):
            _layer_body(h, a, bt, w1, b1, w2, b2, gw, gb, gm, out_ref,
                        hmlp, ssum, ssq, scnt, R=R, H=H, final=False,
                        agg_partial=agg_partial)

    return pl.pallas_call(
        body,
        grid=grid,
        in_specs=in_specs,
        out_specs=out_spec,
        out_shape=out_shape,
        scratch_shapes=scratch,
        compiler_params=pltpu.CompilerParams(
            dimension_semantics=("arbitrary", "arbitrary")),
    )


def kernel(x, edge_index, batch, W1_0, b1_0, W2_0, b2_0, gw_0, gb_0, gm_0,
           W1_1, b1_1, W2_1, b2_1, gw_1, gb_1, gm_1,
           W1_2, b1_2, W2_2, b2_2, gw_2, gb_2, gm_2, Wm1, bm1, Wm2, bm2):
    N, F_IN = x.shape
    E = edge_index.shape[1]
    H = W2_0.shape[0]
    C = Wm2.shape[1]
    R = 2000
    NB = N // R

    src = edge_index[0]
    dst = edge_index[1]
    srcs2 = jnp.concatenate([src, src + N])
    batch3 = batch.reshape(NB, 1, R)

    def row(v):
        return v.reshape(1, -1)

    sc0 = _make_sc_agg(N, E, F_IN, edge_split=True)
    sc128 = _make_sc_agg(N, E, H // 2)
    z128 = jnp.zeros((N, H // 2), jnp.float32)

    tc0 = _make_tc_layer(N, F_IN, H, R, final=False, agg_partial=True)
    tc1 = _make_tc_layer(N, H, H, R, final=False)
    tc2 = _make_tc_layer(N, H, H, R, final=True, C=C)

    agg0 = sc0(x, src, dst, jnp.zeros((N, F_IN), jnp.float32))
    h1 = tc0(x, agg0.reshape(2, N, F_IN), batch3,
             W1_0, row(b1_0), W2_0, row(b2_0), row(gw_0), row(gb_0),
             row(gm_0))
    agg1 = sc128(h1.reshape(2 * N, H // 2), srcs2, dst, z128)
    h2 = tc1(h1, agg1.reshape(2, N, H // 2), batch3,
             W1_1, row(b1_1), W2_1, row(b2_1), row(gw_1), row(gb_1),
             row(gm_1))
    agg2 = sc128(h2.reshape(2 * N, H // 2), srcs2, dst, z128)
    out = tc2(h2, agg2.reshape(2, N, H // 2), batch3,
              W1_2, row(b1_2), W2_2, row(b2_2), row(gw_2), row(gb_2),
              row(gm_2), Wm1, row(bm1), Wm2, row(bm2))
    return out
